# baseline (speedup 1.0000x reference)
import functools

import jax
import jax.numpy as jnp
from jax import lax
from jax.experimental import pallas as pl
from jax.experimental.pallas import tpu as pltpu
from jax.experimental.pallas import tpu_sc as plsc

_LANES = 128


def _tc1_body(x_ref, neurons_ref, dist2_ref, bmu_ref):
    x = x_ref[...]
    n = neurons_ref[...]
    g = lax.dot_general(x, n, (((1,), (1,)), ((), ())),
                        precision=lax.Precision.HIGHEST,
                        preferred_element_type=jnp.float32)
    x2 = jnp.sum(x * x, axis=1, keepdims=True)
    n2 = jnp.sum(n * n, axis=1, keepdims=True).T
    dist2 = jnp.maximum(x2 + n2 - 2.0 * g, 0.0)
    m = jnp.min(dist2, axis=1, keepdims=True)
    col = lax.broadcasted_iota(jnp.int32, dist2.shape, 1)
    bmu = jnp.min(jnp.where(dist2 == m, col, jnp.int32(dist2.shape[1])),
                  axis=1, keepdims=True)
    dist2_ref[...] = dist2
    bmu_ref[...] = bmu


def _tc2_body(dist2_ref, y_ref, act_ref, cnt_ref, sigma_ref,
              out_act_ref, out_cnt_ref):
    dist2 = dist2_ref[...]
    m = jnp.min(dist2, axis=1, keepdims=True)
    sigma = sigma_ref[0, 0]
    gauss = jnp.exp(-jnp.sqrt(dist2) / sigma)
    bmu_g = jnp.exp(-jnp.sqrt(m) / sigma)
    norm = jnp.where(bmu_g == 0.0, jnp.zeros_like(gauss), gauss / bmu_g)
    y = y_ref[...]
    out_act_ref[...] = act_ref[...] + lax.dot_general(
        norm, y, (((0,), (0,)), ((), ())), preferred_element_type=jnp.float32)
    out_cnt_ref[...] = cnt_ref[...] + jnp.sum(y, axis=0,
                                              keepdims=True).astype(jnp.int32)


def _sc_gather_rows(table, idx):
    info = plsc.get_sparse_core_info()
    nw = info.num_cores * info.num_subcores
    B = idx.shape[0]
    Dp = table.shape[1]
    b_per_w = B // nw
    mesh = plsc.VectorSubcoreMesh(core_axis_name="c", subcore_axis_name="s")

    @functools.partial(
        pl.kernel, mesh=mesh,
        out_type=jax.ShapeDtypeStruct((B, Dp), jnp.float32),
        scratch_types=[
            pltpu.VMEM((b_per_w,), jnp.int32),
            pltpu.VMEM((b_per_w, Dp), jnp.float32),
            pltpu.SemaphoreType.DMA,
        ],
    )
    def k(table_hbm, idx_hbm, out_hbm, idx_v, rows_v, sem):
        wid = lax.axis_index("s") * info.num_cores + lax.axis_index("c")
        base = wid * b_per_w
        pltpu.sync_copy(idx_hbm.at[pl.ds(base, b_per_w)], idx_v)
        pltpu.async_copy(table_hbm.at[idx_v], rows_v, sem).wait()
        pltpu.sync_copy(rows_v, out_hbm.at[pl.ds(base, b_per_w)])

    return k(table, idx)


def kernel(x, y, neurons, labels, activities, labels_count, sigma):
    B = x.shape[0]
    H, W, C = labels.shape
    N = H * W
    cnt2d = labels_count.reshape(1, C)
    sigma2d = jnp.asarray(sigma, jnp.float32).reshape(1, 1)
    dist2, bmu2d = pl.pallas_call(
        _tc1_body,
        out_shape=[
            jax.ShapeDtypeStruct((B, N), jnp.float32),
            jax.ShapeDtypeStruct((B, 1), jnp.int32),
        ],
        in_specs=[pl.BlockSpec(memory_space=pltpu.VMEM)] * 2,
        out_specs=[pl.BlockSpec(memory_space=pltpu.VMEM)] * 2,
    )(x, neurons)
    labels_pad = jnp.pad(labels.reshape(N, C), ((0, 0), (0, _LANES - C)))
    out = _sc_gather_rows(labels_pad, bmu2d.reshape(B))[:, :C]
    new_act, new_cnt = pl.pallas_call(
        _tc2_body,
        out_shape=[
            jax.ShapeDtypeStruct((N, C), jnp.float32),
            jax.ShapeDtypeStruct((1, C), jnp.int32),
        ],
        in_specs=[pl.BlockSpec(memory_space=pltpu.VMEM)] * 4
        + [pl.BlockSpec(memory_space=pltpu.SMEM)],
        out_specs=[pl.BlockSpec(memory_space=pltpu.VMEM)] * 2,
    )(dist2, y, activities, cnt2d, sigma2d)
    return out, new_act, new_cnt.reshape(C)


# speedup vs baseline: 1.1877x; 1.1877x over previous
import functools

import jax
import jax.numpy as jnp
from jax import lax
from jax.experimental import pallas as pl
from jax.experimental.pallas import tpu as pltpu
from jax.experimental.pallas import tpu_sc as plsc

_LANES = 128


def _tc_body(x_ref, y_ref, neurons_ref, act_ref, cnt_ref, sigma_ref,
             out_act_ref, out_cnt_ref, bmu_ref):
    x = x_ref[...]
    n = neurons_ref[...]
    g = lax.dot_general(x, n, (((1,), (1,)), ((), ())),
                        precision=lax.Precision.HIGHEST,
                        preferred_element_type=jnp.float32)
    x2 = jnp.sum(x * x, axis=1, keepdims=True)
    n2 = jnp.sum(n * n, axis=1, keepdims=True).T
    dist2 = jnp.maximum(x2 + n2 - 2.0 * g, 0.0)
    m = jnp.min(dist2, axis=1, keepdims=True)
    col = lax.broadcasted_iota(jnp.int32, dist2.shape, 1)
    bmu = jnp.min(jnp.where(dist2 == m, col, jnp.int32(dist2.shape[1])),
                  axis=1, keepdims=True)
    sigma = sigma_ref[0, 0]
    gauss = jnp.exp(-jnp.sqrt(dist2) / sigma)
    bmu_g = jnp.exp(-jnp.sqrt(m) / sigma)
    norm = jnp.where(bmu_g == 0.0, jnp.zeros_like(gauss), gauss / bmu_g)
    y = y_ref[...]
    out_act_ref[...] = act_ref[...] + lax.dot_general(
        norm, y, (((0,), (0,)), ((), ())), preferred_element_type=jnp.float32)
    out_cnt_ref[...] = cnt_ref[...] + jnp.sum(y, axis=0,
                                              keepdims=True).astype(jnp.int32)
    bmu_ref[...] = bmu.T


def _tc_call(x, y, neurons, activities, cnt2d, sigma2d):
    B = x.shape[0]
    N, C = activities.shape
    return pl.pallas_call(
        _tc_body,
        out_shape=[
            jax.ShapeDtypeStruct((N, C), jnp.float32),
            jax.ShapeDtypeStruct((1, C), jnp.int32),
            jax.ShapeDtypeStruct((1, B), jnp.int32),
        ],
        in_specs=[pl.BlockSpec(memory_space=pltpu.VMEM)] * 5
        + [pl.BlockSpec(memory_space=pltpu.SMEM)],
        out_specs=[pl.BlockSpec(memory_space=pltpu.VMEM)] * 3,
    )(x, y, neurons, activities, cnt2d, sigma2d)


def _sc_gather_rows(table, idx):
    info = plsc.get_sparse_core_info()
    nw = info.num_subcores
    B = idx.shape[0]
    Dp = table.shape[1]
    b_per_w = B // nw
    mesh = plsc.VectorSubcoreMesh(core_axis_name="c", subcore_axis_name="s", num_cores=1)

    @functools.partial(
        pl.kernel, mesh=mesh,
        out_type=jax.ShapeDtypeStruct((B, Dp), jnp.float32),
        scratch_types=[
            pltpu.VMEM((b_per_w,), jnp.int32),
            pltpu.VMEM((b_per_w, Dp), jnp.float32),
            pltpu.SemaphoreType.DMA,
        ],
    )
    def k(table_hbm, idx_hbm, out_hbm, idx_v, rows_v, sem):
        wid = lax.axis_index("s")
        base = wid * b_per_w
        pltpu.sync_copy(idx_hbm.at[pl.ds(base, b_per_w)], idx_v)
        pltpu.async_copy(table_hbm.at[idx_v], rows_v, sem).wait()
        pltpu.sync_copy(rows_v, out_hbm.at[pl.ds(base, b_per_w)])

    return k(table, idx)


def kernel(x, y, neurons, labels, activities, labels_count, sigma):
    B = x.shape[0]
    H, W, C = labels.shape
    N = H * W
    cnt2d = labels_count.reshape(1, C)
    sigma2d = jnp.asarray(sigma, jnp.float32).reshape(1, 1)
    labels_pad = jnp.pad(labels.reshape(N, C), ((0, 0), (0, _LANES - C)))
    new_act, new_cnt, bmu2d = _tc_call(x, y, neurons, activities, cnt2d, sigma2d)
    out = _sc_gather_rows(labels_pad, bmu2d.reshape(B))[:, :C]
    return out, new_act, new_cnt.reshape(C)
